# baseline (speedup 1.0000x reference)
import jax
import jax.numpy as jnp
from jax import lax
from jax.experimental import pallas as pl
from jax.experimental.pallas import tpu as pltpu
from jax.experimental.pallas import tpu_sc as plsc

B = 16384
V = 1000000
D = 64
H = 128

NS = 16
CW = 512
NCHK = V // CW
NG = B // 16
NSLOT = 64


def _stream_table(t_hbm, tidx_hbm, tout_hbm, sid,
                  idx_v, midx_v, mk_v, chunk_v, stage_v, hcol_v, hk_v,
                  gsem0, gsem1, wsem):
    kloc = lax.iota(jnp.int32, 16)
    gsems = (gsem0, gsem1)

    pltpu.sync_copy(tidx_hbm.at[pl.ds(0, B)], idx_v)

    def pre(g, off):
        iv = idx_v[pl.ds(g * 16, 16)]
        mask = ((iv >> 9) & 15) == sid
        plsc.store_compressed(midx_v.at[pl.ds(off, 16)], iv, mask=mask)
        plsc.store_compressed(mk_v.at[pl.ds(off, 16)], g * 16 + kloc,
                              mask=mask)
        return off + jnp.max(plsc.all_reduce_population_count(mask))

    n_own = lax.fori_loop(0, NG, pre, jnp.int32(0))
    ng = (n_own + 15) >> 4

    def start_chunk(j, b):
        @pl.when(j < NCHK)
        def _():
            col0 = pl.multiple_of(j * CW, CW)
            colh = pl.multiple_of(j * CW + CW // 2, CW // 2)
            pltpu.async_copy(t_hbm.at[:, pl.ds(col0, CW // 2)],
                             chunk_v.at[b, :, pl.ds(0, CW // 2)], gsems[b])
            pltpu.async_copy(t_hbm.at[:, pl.ds(colh, CW // 2)],
                             chunk_v.at[b, :, pl.ds(CW // 2, CW // 2)],
                             gsems[b])

        @pl.when(j == NCHK)
        def _():
            col0 = pl.multiple_of(j * CW, 128)
            pltpu.async_copy(t_hbm.at[:, pl.ds(col0, 128)],
                             chunk_v.at[b, :, pl.ds(0, 128)], gsems[b])

    def wait_chunk(j, b):
        @pl.when(j < NCHK)
        def _():
            col0 = pl.multiple_of(j * CW, CW)
            pltpu.make_async_copy(t_hbm.at[:, pl.ds(col0, CW)],
                                  chunk_v.at[b], gsems[b]).wait()

        @pl.when(j == NCHK)
        def _():
            col0 = pl.multiple_of(j * CW, 128)
            pltpu.make_async_copy(t_hbm.at[:, pl.ds(col0, 128)],
                                  chunk_v.at[b, :, pl.ds(0, 128)],
                                  gsems[b]).wait()

    for b in range(2):
        start_chunk(jnp.int32(b * NS) + sid, b)

    def make_body(b):
        def dohit(h, nh):
            hm = kloc == h
            hcv = hcol_v[pl.ds(0, 16)]
            hkv = hk_v[pl.ds(0, 16)]
            col = jnp.max(jnp.where(hm, hcv, 0))
            k = jnp.max(jnp.where(hm, hkv, 0))
            colv = jnp.full((16,), col, jnp.int32)
            slot = nh & (NSLOT - 1)
            slotv = jnp.full((16,), slot, jnp.int32)
            for q in range(4):
                cvec = kloc + q * 16
                v = plsc.load_gather(chunk_v.at[b], [cvec, colv])
                plsc.store_scatter(stage_v, [slotv, q * 16 + kloc], v)
            pltpu.async_copy(stage_v.at[pl.ds(slot, 1)],
                             tout_hbm.at[pl.ds(k, 1)], wsem)

            @pl.when(slot == NSLOT - 1)
            def _():
                def dr(i, _):
                    pltpu.make_async_copy(
                        stage_v.at[pl.ds(0, 1)],
                        tout_hbm.at[pl.ds(0, 1)], wsem).wait()
                    return ()

                lax.fori_loop(0, NSLOT, dr, ())

            return nh + 1

        return dohit

    def pair(jj2, nh):
        for b in range(2):
            jj = 2 * jj2 + b
            j = jj * NS + sid
            dohit = make_body(b)

            def scan(g, nh):
                miv = midx_v[pl.ds(g * 16, 16)]
                hit = (miv >> 9) == j
                npc = jnp.max(plsc.all_reduce_population_count(hit))

                def hits(nh):
                    mkv = mk_v[pl.ds(g * 16, 16)]
                    plsc.store_compressed(hcol_v.at[pl.ds(0, 16)],
                                          miv & (CW - 1), mask=hit)
                    plsc.store_compressed(hk_v.at[pl.ds(0, 16)], mkv,
                                          mask=hit)
                    return lax.fori_loop(0, npc, dohit, nh)

                return lax.cond(npc > 0, hits, lambda x: x, nh)

            def active(nh):
                wait_chunk(j, b)
                nh = lax.fori_loop(0, ng, scan, nh)
                start_chunk(j + 2 * NS, b)
                return nh

            nh = lax.cond(j <= NCHK, active, lambda x: x, nh)
        return nh

    nh = lax.fori_loop(0, 62, pair, jnp.int32(0))

    def drain_rest(i, _):
        pltpu.make_async_copy(stage_v.at[pl.ds(0, 1)],
                              tout_hbm.at[pl.ds(0, 1)], wsem).wait()
        return ()

    lax.fori_loop(0, nh & (NSLOT - 1), drain_rest, ())


def _gather_body(ut_hbm, it_hbm, uidx_hbm, iidx_hbm, uout_hbm, iout_hbm,
                 idx_v, midx_v, mk_v, chunk_v, stage_v, hcol_v, hk_v,
                 gsem0, gsem1, wsem):
    cc = lax.axis_index("c")
    sid = lax.axis_index("s")
    scratch = (idx_v, midx_v, mk_v, chunk_v, stage_v, hcol_v, hk_v,
               gsem0, gsem1, wsem)

    @pl.when(cc == 0)
    def _():
        _stream_table(ut_hbm, uidx_hbm, uout_hbm, sid, *scratch)

    @pl.when(cc == 1)
    def _():
        _stream_table(it_hbm, iidx_hbm, iout_hbm, sid, *scratch)


def _sc_gather(uT, iT, uidx, iidx):
    mesh = plsc.VectorSubcoreMesh(core_axis_name="c", subcore_axis_name="s")
    f = pl.kernel(
        _gather_body,
        mesh=mesh,
        compiler_params=pltpu.CompilerParams(needs_layout_passes=False),
        out_type=[
            jax.ShapeDtypeStruct((B, D), jnp.float32),
            jax.ShapeDtypeStruct((B, D), jnp.float32),
        ],
        scratch_types=[
            pltpu.VMEM((B,), jnp.int32),
            pltpu.VMEM((B,), jnp.int32),
            pltpu.VMEM((B,), jnp.int32),
            pltpu.VMEM((2, D, CW), jnp.float32),
            pltpu.VMEM((NSLOT, D), jnp.float32),
            pltpu.VMEM((16,), jnp.int32),
            pltpu.VMEM((16,), jnp.int32),
            pltpu.SemaphoreType.DMA,
            pltpu.SemaphoreType.DMA,
            pltpu.SemaphoreType.DMA,
        ],
    )
    return f(uT, iT, uidx, iidx)


BLK = 2048


def _mlp_body(xu_ref, xi_ref, wu1, bu1, wu2, bu2, wi1, bi1, wi2, bi2,
              ou_ref, oi_ref):
    def tower(x, W1, b1, W2, b2, o_ref):
        h = jnp.maximum(
            jnp.dot(x, W1, preferred_element_type=jnp.float32) + b1, 0.0)
        y = jnp.dot(h, W2, preferred_element_type=jnp.float32) + b2
        n = jnp.sqrt(jnp.sum(y * y, axis=1, keepdims=True))
        o_ref[...] = y / jnp.maximum(n, 1e-12)

    tower(xu_ref[...], wu1[...], bu1[...], wu2[...], bu2[...], ou_ref)
    tower(xi_ref[...], wi1[...], bi1[...], wi2[...], bi2[...], oi_ref)


def _tc_mlp(xu, xi, Wu1, bu1, Wu2, bu2, Wi1, bi1, Wi2, bi2):
    grid = (B // BLK,)
    xspec = pl.BlockSpec((BLK, D), lambda i: (i, 0))
    full = lambda shape: pl.BlockSpec(shape, lambda i: (0, 0))
    return pl.pallas_call(
        _mlp_body,
        grid=grid,
        in_specs=[
            xspec, xspec,
            full((D, H)), full((1, H)), full((H, D)), full((1, D)),
            full((D, H)), full((1, H)), full((H, D)), full((1, D)),
        ],
        out_specs=[xspec, xspec],
        out_shape=[
            jax.ShapeDtypeStruct((B, D), jnp.float32),
            jax.ShapeDtypeStruct((B, D), jnp.float32),
        ],
    )(xu, xi, Wu1, bu1.reshape(1, H), Wu2, bu2.reshape(1, D),
      Wi1, bi1.reshape(1, H), Wi2, bi2.reshape(1, D))


def kernel(user_ids, item_ids, user_table, item_table,
           Wu1, bu1, Wu2, bu2, Wi1, bi1, Wi2, bi2):
    u_pool, i_pool = _sc_gather(user_table.T, item_table.T,
                                user_ids.astype(jnp.int32),
                                item_ids.astype(jnp.int32))
    u_emb, i_emb = _tc_mlp(u_pool, i_pool,
                           Wu1, bu1, Wu2, bu2, Wi1, bi1, Wi2, bi2)
    return (u_emb, i_emb)


# speedup vs baseline: 1.0532x; 1.0532x over previous
import jax
import jax.numpy as jnp
from jax import lax
from jax.experimental import pallas as pl
from jax.experimental.pallas import tpu as pltpu
from jax.experimental.pallas import tpu_sc as plsc

B = 16384
V = 1000000
D = 64
H = 128

NS = 16
CW = 512
NCHK = V // CW
NG = B // 16
NSLOT = 64


def _stream_table(t_hbm, tidx_hbm, tout_hbm, sid,
                  idx_v, midx_v, mk_v, chunk_v, stage_v, hcol_v, hk_v,
                  gsem0, gsem1, wsem):
    kloc = lax.iota(jnp.int32, 16)
    gsems = (gsem0, gsem1)

    pltpu.sync_copy(tidx_hbm.at[pl.ds(0, B)], idx_v)

    def pre(g, off):
        iv = idx_v[pl.ds(g * 16, 16)]
        mask = ((iv >> 9) & 15) == sid
        plsc.store_compressed(midx_v.at[pl.ds(off, 16)], iv, mask=mask)
        plsc.store_compressed(mk_v.at[pl.ds(off, 16)], g * 16 + kloc,
                              mask=mask)
        return off + jnp.max(plsc.all_reduce_population_count(mask))

    n_own = lax.fori_loop(0, NG, pre, jnp.int32(0))
    ng = (n_own + 15) >> 4

    def start_chunk(j, b):
        @pl.when(j < NCHK)
        def _():
            col0 = pl.multiple_of(j * CW, CW)
            pltpu.async_copy(t_hbm.at[:, pl.ds(col0, CW)], chunk_v.at[b],
                             gsems[b])

        @pl.when(j == NCHK)
        def _():
            col0 = pl.multiple_of(j * CW, 128)
            pltpu.async_copy(t_hbm.at[:, pl.ds(col0, 128)],
                             chunk_v.at[b, :, pl.ds(0, 128)], gsems[b])

    def wait_chunk(j, b):
        @pl.when(j < NCHK)
        def _():
            col0 = pl.multiple_of(j * CW, CW)
            pltpu.make_async_copy(t_hbm.at[:, pl.ds(col0, CW)],
                                  chunk_v.at[b], gsems[b]).wait()

        @pl.when(j == NCHK)
        def _():
            col0 = pl.multiple_of(j * CW, 128)
            pltpu.make_async_copy(t_hbm.at[:, pl.ds(col0, 128)],
                                  chunk_v.at[b, :, pl.ds(0, 128)],
                                  gsems[b]).wait()

    for b in range(2):
        start_chunk(jnp.int32(b * NS) + sid, b)

    def make_body(b):
        def dohit(h, nh):
            hm = kloc == h
            hcv = hcol_v[pl.ds(0, 16)]
            hkv = hk_v[pl.ds(0, 16)]
            col = jnp.max(jnp.where(hm, hcv, 0))
            k = jnp.max(jnp.where(hm, hkv, 0))
            colv = jnp.full((16,), col, jnp.int32)
            slot = nh & (NSLOT - 1)
            slotv = jnp.full((16,), slot, jnp.int32)
            for q in range(4):
                cvec = kloc + q * 16
                v = plsc.load_gather(chunk_v.at[b], [cvec, colv])
                plsc.store_scatter(stage_v, [slotv, q * 16 + kloc], v)
            pltpu.async_copy(stage_v.at[pl.ds(slot, 1)],
                             tout_hbm.at[pl.ds(k, 1)], wsem)

            @pl.when(slot == NSLOT - 1)
            def _():
                def dr(i, _):
                    pltpu.make_async_copy(
                        stage_v.at[pl.ds(0, 1)],
                        tout_hbm.at[pl.ds(0, 1)], wsem).wait()
                    return ()

                lax.fori_loop(0, NSLOT, dr, ())

            return nh + 1

        return dohit

    def pair(jj2, nh):
        for b in range(2):
            jj = 2 * jj2 + b
            j = jj * NS + sid
            dohit = make_body(b)

            def scan(g, nh):
                miv = midx_v[pl.ds(g * 16, 16)]
                hit = (miv >> 9) == j
                npc = jnp.max(plsc.all_reduce_population_count(hit))

                def hits(nh):
                    mkv = mk_v[pl.ds(g * 16, 16)]
                    plsc.store_compressed(hcol_v.at[pl.ds(0, 16)],
                                          miv & (CW - 1), mask=hit)
                    plsc.store_compressed(hk_v.at[pl.ds(0, 16)], mkv,
                                          mask=hit)
                    return lax.fori_loop(0, npc, dohit, nh)

                return lax.cond(npc > 0, hits, lambda x: x, nh)

            def active(nh):
                wait_chunk(j, b)
                nh = lax.fori_loop(0, ng, scan, nh)
                start_chunk(j + 2 * NS, b)
                return nh

            nh = lax.cond(j <= NCHK, active, lambda x: x, nh)
        return nh

    nh = lax.fori_loop(0, 62, pair, jnp.int32(0))

    def drain_rest(i, _):
        pltpu.make_async_copy(stage_v.at[pl.ds(0, 1)],
                              tout_hbm.at[pl.ds(0, 1)], wsem).wait()
        return ()

    lax.fori_loop(0, nh & (NSLOT - 1), drain_rest, ())


def _gather_body(ut_hbm, it_hbm, uidx_hbm, iidx_hbm, uout_hbm, iout_hbm,
                 idx_v, midx_v, mk_v, chunk_v, stage_v, hcol_v, hk_v,
                 gsem0, gsem1, wsem):
    cc = lax.axis_index("c")
    sid = lax.axis_index("s")
    scratch = (idx_v, midx_v, mk_v, chunk_v, stage_v, hcol_v, hk_v,
               gsem0, gsem1, wsem)

    @pl.when(cc == 0)
    def _():
        _stream_table(ut_hbm, uidx_hbm, uout_hbm, sid, *scratch)

    @pl.when(cc == 1)
    def _():
        _stream_table(it_hbm, iidx_hbm, iout_hbm, sid, *scratch)


def _sc_gather(uT, iT, uidx, iidx):
    mesh = plsc.VectorSubcoreMesh(core_axis_name="c", subcore_axis_name="s")
    f = pl.kernel(
        _gather_body,
        mesh=mesh,
        compiler_params=pltpu.CompilerParams(needs_layout_passes=False),
        out_type=[
            jax.ShapeDtypeStruct((B, D), jnp.float32),
            jax.ShapeDtypeStruct((B, D), jnp.float32),
        ],
        scratch_types=[
            pltpu.VMEM((B,), jnp.int32),
            pltpu.VMEM((B,), jnp.int32),
            pltpu.VMEM((B,), jnp.int32),
            pltpu.VMEM((2, D, CW), jnp.float32),
            pltpu.VMEM((NSLOT, D), jnp.float32),
            pltpu.VMEM((16,), jnp.int32),
            pltpu.VMEM((16,), jnp.int32),
            pltpu.SemaphoreType.DMA,
            pltpu.SemaphoreType.DMA,
            pltpu.SemaphoreType.DMA,
        ],
    )
    return f(uT, iT, uidx, iidx)


BLK = 2048


_DN_XT = (((0,), (1,)), ((), ()))
_DN_00 = (((0,), (0,)), ((), ()))


def _mlp_body(xu_ref, xi_ref, wu1, bu1, wu2, bu2, wi1, bi1, wi2, bi2,
              ou_ref, oi_ref):
    def tower(x, W1, b1, W2, b2, o_ref):
        hT = jnp.maximum(
            lax.dot_general(W1, x, _DN_XT,
                            preferred_element_type=jnp.float32) + b1, 0.0)
        yT = lax.dot_general(W2, hT, _DN_00,
                             preferred_element_type=jnp.float32) + b2
        n = jnp.sqrt(jnp.sum(yT * yT, axis=0, keepdims=True))
        o_ref[...] = yT / jnp.maximum(n, 1e-12)

    tower(xu_ref[...], wu1[...], bu1[...], wu2[...], bu2[...], ou_ref)
    tower(xi_ref[...], wi1[...], bi1[...], wi2[...], bi2[...], oi_ref)


def _tc_mlp(xu, xi, Wu1, bu1, Wu2, bu2, Wi1, bi1, Wi2, bi2):
    grid = (B // BLK,)
    xspec = pl.BlockSpec((BLK, D), lambda i: (i, 0))
    ospec = pl.BlockSpec((D, BLK), lambda i: (0, i))
    full = lambda shape: pl.BlockSpec(shape, lambda i: (0, 0))
    return pl.pallas_call(
        _mlp_body,
        grid=grid,
        in_specs=[
            xspec, xspec,
            full((D, H)), full((H, 1)), full((H, D)), full((D, 1)),
            full((D, H)), full((H, 1)), full((H, D)), full((D, 1)),
        ],
        out_specs=[ospec, ospec],
        out_shape=[
            jax.ShapeDtypeStruct((D, B), jnp.float32),
            jax.ShapeDtypeStruct((D, B), jnp.float32),
        ],
    )(xu, xi, Wu1, bu1.reshape(H, 1), Wu2, bu2.reshape(D, 1),
      Wi1, bi1.reshape(H, 1), Wi2, bi2.reshape(D, 1))


def kernel(user_ids, item_ids, user_table, item_table,
           Wu1, bu1, Wu2, bu2, Wi1, bi1, Wi2, bi2):
    u_pool, i_pool = _sc_gather(user_table.T, item_table.T,
                                user_ids.astype(jnp.int32),
                                item_ids.astype(jnp.int32))
    uT_emb, iT_emb = _tc_mlp(u_pool, i_pool,
                             Wu1, bu1, Wu2, bu2, Wi1, bi1, Wi2, bi2)
    return (uT_emb.T, iT_emb.T)
